# Initial kernel scaffold; baseline (speedup 1.0000x reference)
import jax
import jax.numpy as jnp
from jax.experimental import pallas as pl


def kernel(group_user, group_behavior_ids, group_behavior_counts, target_user, similarity_vec, user_table, item_table):
    raise NotImplementedError("write your pallas kernel here")


# speedup vs baseline: 12.6300x; 12.6300x over previous
import functools

import jax
import jax.numpy as jnp
from jax import lax
from jax.experimental import pallas as pl
from jax.experimental.pallas import tpu as pltpu
from jax.experimental.pallas import tpu_sc as plsc

G = 1024
U = 20
L = 50
D = 64
FACTOR = 0.5

NC = 2
NS = 16
NW = NC * NS
GPW = G // NW

CHUNK = 100
NCHUNK = (U * L) // CHUNK
NBUF = 3
NK = D // 16


def _body(gbi_hbm, cnt_hbm, gu_hbm, tgt_hbm, sim_hbm, utab_hbm, itab_hbm,
          out_hbm,
          idx_v, cnt_v, rows_v, gu_v, tgt_v, trows_v, urows_v, srows_v,
          out_v, sem0, sem1, sem2, semg):
    sems = [sem0, sem1, sem2]
    wid = lax.axis_index("s") * NC + lax.axis_index("c")
    gbase = wid * GPW

    pltpu.sync_copy(gu_hbm.at[pl.ds(gbase, GPW)], gu_v)
    pltpu.sync_copy(tgt_hbm.at[pl.ds(gbase, GPW)], tgt_v)
    pltpu.async_copy(sim_hbm.at[tgt_v], trows_v, semg).wait()

    @pl.loop(0, GPW)
    def _group(gi):
        g = gbase + gi
        pltpu.sync_copy(gbi_hbm.at[g], idx_v)
        pltpu.sync_copy(cnt_hbm.at[g], cnt_v.at[pl.ds(0, U * L)])
        cp_u = pltpu.async_copy(utab_hbm.at[gu_v.at[gi]], urows_v, semg)
        cp_s = pltpu.async_copy(sim_hbm.at[gu_v.at[gi]], srows_v, semg)
        cps = [None] * NCHUNK
        for j in range(NBUF):
            cps[j] = pltpu.async_copy(itab_hbm.at[idx_v.at[j]],
                                      rows_v.at[j], sems[j])
        ts = [trows_v[gi, pl.ds(16 * k, 16)] for k in range(NK)]
        cp_u.wait()
        cp_s.wait()

        og = [jnp.zeros((16,), jnp.float32) for _ in range(NK)]
        for j in range(NCHUNK):
            b = j % NBUF
            cps[j].wait()
            for u2 in range(2):
                u = 2 * j + u2

                def _lbody(l, acc):
                    c = cnt_v[pl.ds(u * L + l, 16)][0]
                    return tuple(
                        acc[k] + rows_v[b, u2 * L + l, pl.ds(16 * k, 16)] * c
                        for k in range(NK))

                acc = lax.fori_loop(
                    0, L, _lbody,
                    tuple(jnp.zeros((16,), jnp.float32) for _ in range(NK)))

                s = jnp.float32(0.0)
                for k in range(NK):
                    s = s + jnp.sum(ts[k] * srows_v[u, pl.ds(16 * k, 16)])
                s = s * FACTOR
                for k in range(NK):
                    og[k] = og[k] + acc[k] * urows_v[u, pl.ds(16 * k, 16)] * s
            if j + NBUF < NCHUNK:
                cps[j + NBUF] = pltpu.async_copy(
                    itab_hbm.at[idx_v.at[j + NBUF]], rows_v.at[b], sems[b])
        for k in range(NK):
            out_v[gi, pl.ds(16 * k, 16)] = og[k]

    pltpu.sync_copy(out_v, out_hbm.at[pl.ds(gbase, GPW)])


_sc_call = pl.kernel(
    _body,
    out_type=jax.ShapeDtypeStruct((G, D), jnp.float32),
    mesh=plsc.VectorSubcoreMesh(core_axis_name="c", subcore_axis_name="s",
                                num_cores=NC, num_subcores=NS),
    compiler_params=pltpu.CompilerParams(needs_layout_passes=False,
                                         use_tc_tiling_on_sc=False),
    scratch_types=[
        pltpu.VMEM((NCHUNK, CHUNK), jnp.int32),
        pltpu.VMEM((U * L + 16,), jnp.float32),
        pltpu.VMEM((NBUF, CHUNK, D), jnp.float32),
        pltpu.VMEM((GPW, U), jnp.int32),
        pltpu.VMEM((GPW,), jnp.int32),
        pltpu.VMEM((GPW, D), jnp.float32),
        pltpu.VMEM((U, D), jnp.float32),
        pltpu.VMEM((U, D), jnp.float32),
        pltpu.VMEM((GPW, D), jnp.float32),
        pltpu.SemaphoreType.DMA,
        pltpu.SemaphoreType.DMA,
        pltpu.SemaphoreType.DMA,
        pltpu.SemaphoreType.DMA,
    ],
)


@jax.jit
def kernel(group_user, group_behavior_ids, group_behavior_counts,
           target_user, similarity_vec, user_table, item_table):
    gbi = group_behavior_ids.astype(jnp.int32).reshape(G, NCHUNK, CHUNK)
    cnt = group_behavior_counts.reshape(G, U * L)
    gu = group_user.astype(jnp.int32)
    tgt = target_user.astype(jnp.int32)
    return _sc_call(gbi, cnt, gu, tgt, similarity_vec, user_table, item_table)
